# baseline (speedup 1.0000x reference)
import functools

import jax
import jax.numpy as jnp
from jax import lax
from jax.experimental import pallas as pl
from jax.experimental.pallas import tpu as pltpu
from jax.experimental.pallas import tpu_sc as plsc

_ROWS_BUF = 1000
_LANES = 16


@functools.lru_cache(maxsize=None)
def _make_broadcast_kernel(n_rows: int, dim: int):
    info = plsc.get_sparse_core_info()
    nc, ns = info.num_cores, info.num_subcores
    nw = nc * ns
    assert n_rows % nw == 0
    per_w = n_rows // nw
    rows_buf = min(_ROWS_BUF, per_w)
    n_full = per_w // rows_buf
    rem = per_w - n_full * rows_buf
    n_vecs = dim // _LANES

    mesh = plsc.VectorSubcoreMesh(core_axis_name="c", subcore_axis_name="s")

    @functools.partial(
        pl.kernel,
        mesh=mesh,
        out_type=jax.ShapeDtypeStruct((n_rows, dim), jnp.float32),
        scratch_types=[
            pltpu.VMEM((rows_buf, dim), jnp.float32),
            pltpu.SemaphoreType.DMA,
        ],
    )
    def broadcast_kernel(w_hbm, out_hbm, buf, sem):
        wid = lax.axis_index("s") * nc + lax.axis_index("c")
        base = wid * per_w

        pltpu.sync_copy(w_hbm, buf.at[0:1])

        w_vecs = [buf[0, pl.ds(j * _LANES, _LANES)] for j in range(n_vecs)]
        head = min(64, rows_buf)

        def fill_rows(lo, hi, unroll):
            span = hi - lo
            if span <= 0:
                return
            assert span % unroll == 0

            def body(i, carry):
                r0 = lo + i * unroll
                for u in range(unroll):
                    for j in range(n_vecs):
                        buf[r0 + u, pl.ds(j * _LANES, _LANES)] = w_vecs[j]
                return carry

            lax.fori_loop(0, span // unroll, body, 0)

        copies = []
        for r in range(1, min(8, head)):
            for j in range(n_vecs):
                buf[r, pl.ds(j * _LANES, _LANES)] = w_vecs[j]
        fill_rows(8, head, 8)
        copies.append(
            pltpu.async_copy(buf.at[0:head], out_hbm.at[pl.ds(base, head)], sem)
        )
        fill_rows(head, rows_buf, 8)
        if rows_buf > head:
            copies.append(
                pltpu.async_copy(
                    buf.at[head:rows_buf],
                    out_hbm.at[pl.ds(base + head, rows_buf - head)],
                    sem,
                )
            )
        for i in range(1, n_full):
            copies.append(
                pltpu.async_copy(
                    buf, out_hbm.at[pl.ds(base + i * rows_buf, rows_buf)], sem
                )
            )
        if rem:
            copies.append(
                pltpu.async_copy(
                    buf.at[0:rem],
                    out_hbm.at[pl.ds(base + n_full * rows_buf, rem)],
                    sem,
                )
            )
        for c in copies:
            c.wait()

    return broadcast_kernel


def kernel(edge_index, W):
    n_rows = edge_index.shape[1]
    dim = W.shape[1]
    return _make_broadcast_kernel(n_rows, dim)(W.astype(jnp.float32))


# speedup vs baseline: 1.0057x; 1.0057x over previous
import functools

import jax
import jax.numpy as jnp
from jax import lax
from jax.experimental import pallas as pl
from jax.experimental.pallas import tpu as pltpu
from jax.experimental.pallas import tpu_sc as plsc

_ROWS_BUF = 512
_LANES = 16


@functools.lru_cache(maxsize=None)
def _make_broadcast_kernel(n_rows: int, dim: int):
    info = plsc.get_sparse_core_info()
    nc, ns = info.num_cores, info.num_subcores
    nw = nc * ns
    assert n_rows % nw == 0
    per_w = n_rows // nw
    rows_buf = min(_ROWS_BUF, per_w)
    n_full = per_w // rows_buf
    rem = per_w - n_full * rows_buf
    n_vecs = dim // _LANES

    mesh = plsc.VectorSubcoreMesh(core_axis_name="c", subcore_axis_name="s")

    @functools.partial(
        pl.kernel,
        mesh=mesh,
        out_type=jax.ShapeDtypeStruct((n_rows, dim), jnp.float32),
        scratch_types=[
            pltpu.VMEM((rows_buf, dim), jnp.float32),
            pltpu.SemaphoreType.DMA,
        ],
    )
    def broadcast_kernel(w_hbm, out_hbm, buf, sem):
        wid = lax.axis_index("s") * nc + lax.axis_index("c")
        base = wid * per_w

        pltpu.sync_copy(w_hbm, buf.at[0:1])

        w_vecs = [buf[0, pl.ds(j * _LANES, _LANES)] for j in range(n_vecs)]
        head = min(64, rows_buf)

        def fill_rows(lo, hi, unroll):
            span = hi - lo
            if span <= 0:
                return
            assert span % unroll == 0

            def body(i, carry):
                r0 = lo + i * unroll
                for u in range(unroll):
                    for j in range(n_vecs):
                        buf[r0 + u, pl.ds(j * _LANES, _LANES)] = w_vecs[j]
                return carry

            lax.fori_loop(0, span // unroll, body, 0)

        copies = []
        for r in range(1, min(8, head)):
            for j in range(n_vecs):
                buf[r, pl.ds(j * _LANES, _LANES)] = w_vecs[j]
        fill_rows(8, head, 8)
        copies.append(
            pltpu.async_copy(buf.at[0:head], out_hbm.at[pl.ds(base, head)], sem)
        )
        for blk in range(head, rows_buf, head):
            blk_hi = min(blk + head, rows_buf)
            fill_rows(blk, blk_hi, 8)
            copies.append(
                pltpu.async_copy(
                    buf.at[blk:blk_hi],
                    out_hbm.at[pl.ds(base + blk, blk_hi - blk)],
                    sem,
                )
            )
        for i in range(1, n_full):
            copies.append(
                pltpu.async_copy(
                    buf, out_hbm.at[pl.ds(base + i * rows_buf, rows_buf)], sem
                )
            )
        if rem:
            copies.append(
                pltpu.async_copy(
                    buf.at[0:rem],
                    out_hbm.at[pl.ds(base + n_full * rows_buf, rem)],
                    sem,
                )
            )
        for c in copies:
            c.wait()

    return broadcast_kernel


def kernel(edge_index, W):
    n_rows = edge_index.shape[1]
    dim = W.shape[1]
    return _make_broadcast_kernel(n_rows, dim)(W.astype(jnp.float32))


# speedup vs baseline: 1.0255x; 1.0197x over previous
import functools

import jax
import jax.numpy as jnp
from jax import lax
from jax.experimental import pallas as pl
from jax.experimental.pallas import tpu as pltpu
from jax.experimental.pallas import tpu_sc as plsc

_ROWS_BUF = 512
_LANES = 16


@functools.lru_cache(maxsize=None)
def _make_broadcast_kernel(n_rows: int, dim: int):
    info = plsc.get_sparse_core_info()
    nc, ns = info.num_cores, info.num_subcores
    nw = nc * ns
    assert n_rows % nw == 0
    per_w = n_rows // nw
    rows_buf = min(_ROWS_BUF, per_w)
    n_full = per_w // rows_buf
    rem = per_w - n_full * rows_buf
    n_vecs = dim // _LANES

    n_s = n_full * 36 // 100
    n_t = n_full - n_s

    mesh = plsc.VectorSubcoreMesh(core_axis_name="c", subcore_axis_name="s")

    @functools.partial(
        pl.kernel,
        mesh=mesh,
        out_type=jax.ShapeDtypeStruct((n_rows, dim), jnp.float32),
        scratch_types=[
            pltpu.VMEM((rows_buf, dim), jnp.float32),
            pltpu.VMEM_SHARED((rows_buf, dim), jnp.float32),
            pltpu.SemaphoreType.DMA,
        ],
    )
    def broadcast_kernel(w_hbm, out_hbm, buf, shared, sem):
        sid = lax.axis_index("s")
        wid = sid * nc + lax.axis_index("c")
        base = wid * per_w

        pltpu.sync_copy(w_hbm, buf.at[0:1])

        w_vecs = [buf[0, pl.ds(j * _LANES, _LANES)] for j in range(n_vecs)]

        def fill_row(r, carry):
            for j in range(n_vecs):
                buf[r, pl.ds(j * _LANES, _LANES)] = w_vecs[j]
            return carry

        lax.fori_loop(1, rows_buf, fill_row, 0)

        copies = []
        for i in range(n_t):
            copies.append(
                pltpu.async_copy(
                    buf, out_hbm.at[pl.ds(base + i * rows_buf, rows_buf)], sem
                )
            )
        if rem:
            copies.append(
                pltpu.async_copy(
                    buf.at[0:rem],
                    out_hbm.at[pl.ds(base + n_full * rows_buf, rem)],
                    sem,
                )
            )

        @pl.when(sid == 0)
        def _():
            pltpu.sync_copy(buf, shared)

        plsc.subcore_barrier()

        for i in range(n_t, n_full):
            copies.append(
                pltpu.async_copy(
                    shared, out_hbm.at[pl.ds(base + i * rows_buf, rows_buf)], sem
                )
            )
        for c in copies:
            c.wait()

    return broadcast_kernel


def kernel(edge_index, W):
    n_rows = edge_index.shape[1]
    dim = W.shape[1]
    return _make_broadcast_kernel(n_rows, dim)(W.astype(jnp.float32))
